# baseline (speedup 1.0000x reference)
import functools

import jax
import jax.numpy as jnp
from jax import lax
from jax.experimental import pallas as pl
from jax.experimental.pallas import tpu as pltpu
from jax.experimental.pallas import tpu_sc as plsc

N_NODE = 50000
E_RAW = 800000
EMB = 64

NCORES = 2
NSUB = 16
CH = 64
NCHUNK = 782
CCH = 64
NCCHUNK = 782
E_TILE = CH * NCHUNK
E_PAD = E_TILE * NSUB
HALF = N_NODE // NCORES
ACC_ROWS = 25088
ROWS_PER_TILE = ACC_ROWS // NSUB
CNTW = 16

_SC_MESH = plsc.VectorSubcoreMesh(core_axis_name="c", subcore_axis_name="s",
                                  num_cores=NCORES, num_subcores=NSUB)
_SC_PARAMS = pltpu.CompilerParams(use_tc_tiling_on_sc=False,
                                  needs_layout_passes=False)


def _localize(idx_ref, lo, n):
    for j in range(n // 16):
        d = idx_ref[pl.ds(j * 16, 16)] - lo
        ok = (d >= 0) & (d < HALF)
        idx_ref[pl.ds(j * 16, 16)] = jnp.where(ok, d, HALF)


def _sc_edge_body(tl_hbm, tr_hbm, pk_hbm, we_hbm,
                  out_hbm, acc, pkA, pkB, locA, locB, efA, efB,
                  glA, grA, glB, grB, joint, wev,
                  s_pkA, s_pkB, s_glA, s_grA, s_glB, s_grB):
    c = lax.axis_index("c")
    s = lax.axis_index("s")
    lo = c * HALF

    def _zero_row(r, _):
        for g in range(EMB // 16):
            joint[r, pl.ds(g * 16, 16)] = jnp.zeros((16,), jnp.float32)
        return 0
    lax.fori_loop(0, CH, _zero_row, 0)

    row0 = s * ROWS_PER_TILE
    nfull = ROWS_PER_TILE // CH
    rem = ROWS_PER_TILE - nfull * CH
    for k in range(nfull):
        pltpu.sync_copy(joint, acc.at[pl.ds(row0 + k * CH, CH)])
    if rem:
        pltpu.sync_copy(joint.at[pl.ds(0, rem)],
                        acc.at[pl.ds(row0 + nfull * CH, rem)])

    pltpu.sync_copy(we_hbm, wev)
    ws = tuple(wev[pl.ds(g * 16, 16)] for g in range(EMB // 16))

    plsc.subcore_barrier()

    pkrow = s * NCHUNK

    def _issue_pk(buf, sem, g):
        pltpu.async_copy(pk_hbm.at[pkrow + g], buf, sem)

    def _wait_pk(buf, sem):
        pltpu.make_async_copy(pk_hbm.at[0], buf, sem).wait()

    def _prep(pk, loc, ef):
        for j in range(CH // 16):
            d = pk[2, pl.ds(j * 16, 16)] - lo
            ok = (d >= 0) & (d < HALF)
            loc[pl.ds(j * 16, 16)] = jnp.where(ok, d, HALF)
            ef[pl.ds(j * 16, 16)] = plsc.bitcast(pk[3, pl.ds(j * 16, 16)],
                                                 jnp.float32)

    def _start_gathers(pk, gl, gr, sg1, sg2):
        pltpu.async_copy(tl_hbm.at[pk.at[0]], gl, sg1)
        pltpu.async_copy(tr_hbm.at[pk.at[1]], gr, sg2)

    def _wait_gathers(pk, gl, gr, sg1, sg2):
        pltpu.make_async_copy(tl_hbm.at[pk.at[0]], gl, sg1).wait()
        pltpu.make_async_copy(tr_hbm.at[pk.at[1]], gr, sg2).wait()

    def _compute(gl, gr, ef, ws):
        def _edge16(j, ws):
            efvec = ef[pl.ds(j * 16, 16)]
            for k in range(16):
                e = j * 16 + k
                efs = efvec[k]
                for g, wg in enumerate(ws):
                    v = gl[e, pl.ds(g * 16, 16)] + gr[e, pl.ds(g * 16, 16)]
                    v = jnp.maximum(v + efs * wg, 0.0)
                    joint[e, pl.ds(g * 16, 16)] = v
            return ws
        return lax.fori_loop(0, CH // 16, _edge16, ws)

    _issue_pk(pkA, s_pkA, 0)
    _issue_pk(pkB, s_pkB, 1)
    _wait_pk(pkA, s_pkA)
    _prep(pkA, locA, efA)
    _start_gathers(pkA, glA, grA, s_glA, s_grA)
    _wait_pk(pkB, s_pkB)
    _prep(pkB, locB, efB)
    _start_gathers(pkB, glB, grB, s_glB, s_grB)

    def _half(g, pk, s_pk, loc, ef, gl, gr, sg1, sg2, ws):
        _wait_gathers(pk, gl, gr, sg1, sg2)

        @pl.when(g + 2 < NCHUNK)
        def _():
            _issue_pk(pk, s_pk, g + 2)

        ws = _compute(gl, gr, ef, ws)
        pltpu.sync_copy(joint, acc.at[loc], add=True)

        @pl.when(g + 2 < NCHUNK)
        def _():
            _wait_pk(pk, s_pk)
            _prep(pk, loc, ef)
            _start_gathers(pk, gl, gr, sg1, sg2)
        return ws

    def _pair(k, ws):
        ws = _half(2 * k, pkA, s_pkA, locA, efA, glA, grA, s_glA, s_grA, ws)
        ws = _half(2 * k + 1, pkB, s_pkB, locB, efB, glB, grB, s_glB, s_grB,
                   ws)
        return ws

    lax.fori_loop(0, NCHUNK // 2, _pair, ws)

    plsc.subcore_barrier()
    pltpu.sync_copy(acc.at[pl.ds(row0, ROWS_PER_TILE)],
                    out_hbm.at[c, pl.ds(row0, ROWS_PER_TILE)])


_sc_edge_pass = functools.partial(
    pl.kernel,
    out_type=jax.ShapeDtypeStruct((NCORES, ACC_ROWS, EMB), jnp.float32),
    mesh=_SC_MESH,
    compiler_params=_SC_PARAMS,
    scratch_types=[
        pltpu.VMEM_SHARED((ACC_ROWS, EMB), jnp.float32),
        pltpu.VMEM((4, CH), jnp.int32),
        pltpu.VMEM((4, CH), jnp.int32),
        pltpu.VMEM((CH,), jnp.int32),
        pltpu.VMEM((CH,), jnp.int32),
        pltpu.VMEM((CH,), jnp.float32),
        pltpu.VMEM((CH,), jnp.float32),
        pltpu.VMEM((CH, EMB), jnp.float32),
        pltpu.VMEM((CH, EMB), jnp.float32),
        pltpu.VMEM((CH, EMB), jnp.float32),
        pltpu.VMEM((CH, EMB), jnp.float32),
        pltpu.VMEM((CH, EMB), jnp.float32),
        pltpu.VMEM((EMB,), jnp.float32),
        pltpu.SemaphoreType.DMA,
        pltpu.SemaphoreType.DMA,
        pltpu.SemaphoreType.DMA,
        pltpu.SemaphoreType.DMA,
        pltpu.SemaphoreType.DMA,
        pltpu.SemaphoreType.DMA,
    ],
)(_sc_edge_body)


def _sc_count_body(d0_hbm, d1_hbm, out0_hbm, out1_hbm, cnt0, cnt1,
                   l0A, l1A, l0B, l1B, ones, s0A, s1A, s0B, s1B):
    c = lax.axis_index("c")
    s = lax.axis_index("s")
    lo = c * HALF

    one_vec = jnp.where(lax.iota(jnp.int32, 16) == 0,
                        jnp.float32(1.0), jnp.float32(0.0))
    zero_vec = jnp.zeros((16,), jnp.float32)

    def _init_row(r, _):
        ones[r, pl.ds(0, 16)] = one_vec
        return 0
    lax.fori_loop(0, CCH, _init_row, 0)

    row0 = s * ROWS_PER_TILE

    def _zrow(r, _):
        ones[r, pl.ds(0, 16)] = zero_vec
        return 0
    lax.fori_loop(0, CCH, _zrow, 0)
    nfull = ROWS_PER_TILE // CCH
    rem = ROWS_PER_TILE - nfull * CCH
    for acc in (cnt0, cnt1):
        for k in range(nfull):
            pltpu.sync_copy(ones, acc.at[pl.ds(row0 + k * CCH, CCH)])
        pltpu.sync_copy(ones.at[pl.ds(0, rem)],
                        acc.at[pl.ds(row0 + nfull * CCH, rem)])
    lax.fori_loop(0, CCH, _init_row, 0)

    plsc.subcore_barrier()

    ebase = s * E_TILE

    def _issue(g, l0, l1, sem0, sem1):
        base = ebase + g * CCH
        pltpu.async_copy(d0_hbm.at[pl.ds(base, CCH)], l0, sem0)
        pltpu.async_copy(d1_hbm.at[pl.ds(base, CCH)], l1, sem1)

    def _wait(l0, l1, sem0, sem1):
        pltpu.make_async_copy(d0_hbm.at[pl.ds(0, CCH)], l0, sem0).wait()
        pltpu.make_async_copy(d1_hbm.at[pl.ds(0, CCH)], l1, sem1).wait()

    _issue(0, l0A, l1A, s0A, s1A)
    _issue(1, l0B, l1B, s0B, s1B)

    def _half(g, l0, l1, sem0, sem1):
        _wait(l0, l1, sem0, sem1)
        _localize(l0, lo, CCH)
        _localize(l1, lo, CCH)
        pltpu.sync_copy(ones, cnt0.at[l0], add=True)
        pltpu.sync_copy(ones, cnt1.at[l1], add=True)

        @pl.when(g + 2 < NCCHUNK)
        def _():
            _issue(g + 2, l0, l1, sem0, sem1)

    def _pair(k, carry):
        _half(2 * k, l0A, l1A, s0A, s1A)
        _half(2 * k + 1, l0B, l1B, s0B, s1B)
        return carry

    lax.fori_loop(0, NCCHUNK // 2, _pair, 0)

    plsc.subcore_barrier()
    pltpu.sync_copy(cnt0.at[pl.ds(row0, ROWS_PER_TILE)],
                    out0_hbm.at[c, pl.ds(row0, ROWS_PER_TILE)])
    pltpu.sync_copy(cnt1.at[pl.ds(row0, ROWS_PER_TILE)],
                    out1_hbm.at[c, pl.ds(row0, ROWS_PER_TILE)])


_sc_count_pass = functools.partial(
    pl.kernel,
    out_type=(jax.ShapeDtypeStruct((NCORES, ACC_ROWS, CNTW), jnp.float32),
              jax.ShapeDtypeStruct((NCORES, ACC_ROWS, CNTW), jnp.float32)),
    mesh=_SC_MESH,
    compiler_params=_SC_PARAMS,
    scratch_types=[
        pltpu.VMEM_SHARED((ACC_ROWS, CNTW), jnp.float32),
        pltpu.VMEM_SHARED((ACC_ROWS, CNTW), jnp.float32),
        pltpu.VMEM((CCH,), jnp.int32),
        pltpu.VMEM((CCH,), jnp.int32),
        pltpu.VMEM((CCH,), jnp.int32),
        pltpu.VMEM((CCH,), jnp.int32),
        pltpu.VMEM((CCH, CNTW), jnp.float32),
        pltpu.SemaphoreType.DMA,
        pltpu.SemaphoreType.DMA,
        pltpu.SemaphoreType.DMA,
        pltpu.SemaphoreType.DMA,
    ],
)(_sc_count_body)


def _mm(x, w):
    return jnp.dot(x, w, preferred_element_type=jnp.float32)


def _phase1_body(cf, vf, csh, csc, cw1, cb1, cw2, cb2, vsh, vsc, vw1, vb1,
                 vw2, vb2, wl1, bl1, wr1, wr2,
                 cons_o, var_o, tl1_o, tr1_o, tr2_o):
    ch = (cf[...] + csh[...]) * csc[...]
    ch = jnp.maximum(_mm(ch, cw1[...]) + cb1[...], 0.0)
    cons = jnp.maximum(_mm(ch, cw2[...]) + cb2[...], 0.0)
    vh = (vf[...] + vsh[...]) * vsc[...]
    vh = jnp.maximum(_mm(vh, vw1[...]) + vb1[...], 0.0)
    var = jnp.maximum(_mm(vh, vw2[...]) + vb2[...], 0.0)
    cons_o[...] = cons
    var_o[...] = var
    tl1_o[...] = _mm(cons, wl1[...]) + bl1[...]
    tr1_o[...] = _mm(var, wr1[...])
    tr2_o[...] = _mm(var, wr2[...])


def _phase2_body(acc, cnt, cons, wfps, bfps, wo1a, wo1b, bo1, wo2, bo2,
                 wl2, bl2, tl2_o):
    conv = _mm(acc[0], wfps[...]) + cnt[0][:, 0:1] * bfps[...]
    h = jnp.maximum(_mm(conv, wo1a[...]) + _mm(cons[...], wo1b[...])
                    + bo1[...], 0.0)
    cn = jnp.maximum(_mm(h, wo2[...]) + bo2[...], 0.0)
    tl2_o[...] = _mm(cn, wl2[...]) + bl2[...]


def _phase3_body(acc, cnt, var, wfps, bfps, wo1a, wo1b, bo1, wo2, bo2,
                 ow1, ob1, ow2r, y_o):
    conv = _mm(acc[0], wfps[...]) + cnt[0][:, 0:1] * bfps[...]
    h = jnp.maximum(_mm(conv, wo1a[...]) + _mm(var[...], wo1b[...])
                    + bo1[...], 0.0)
    vn = jnp.maximum(_mm(h, wo2[...]) + bo2[...], 0.0)
    z = jnp.maximum(_mm(vn, ow1[...]) + ob1[...], 0.0)
    y_o[...] = jnp.sum(z * ow2r[...], axis=1, keepdims=True)


def _edge_body(e, sh1, sc1, sh2, sc2, ef1_o, ef2_o):
    ev = e[...]
    ef1_o[...] = ev * sc1[0, 0] + sh1[0, 0]
    ef2_o[...] = ev * sc2[0, 0] + sh2[0, 0]


def _full(shape):
    return pl.BlockSpec(shape, lambda i: tuple(0 for _ in shape))


def _smem11():
    return pl.BlockSpec((1, 1), lambda i: (0, 0), memory_space=pltpu.SMEM)


def kernel(constraint_features, edge_indices, edge_features,
           variable_features, n_cons_per_sample, n_vars_per_sample, params):
    p = params
    f32 = jnp.float32

    idx0 = edge_indices[0].astype(jnp.int32)
    idx1 = edge_indices[1].astype(jnp.int32)
    npad = E_PAD - E_RAW
    zpad = jnp.zeros((npad,), jnp.int32)
    gpad = jnp.full((npad,), N_NODE, jnp.int32)
    g0 = jnp.concatenate([idx0, zpad])
    g1 = jnp.concatenate([idx1, zpad])
    d0 = jnp.concatenate([idx0, gpad])
    d1 = jnp.concatenate([idx1, gpad])
    e_flat = jnp.concatenate([edge_features[:, 0].astype(f32),
                              jnp.zeros((npad,), f32)])
    e2d = e_flat.reshape(E_PAD // 128, 128)

    cf = jnp.pad(constraint_features.astype(f32), ((0, 0), (0, 3)))
    vf = jnp.pad(variable_features.astype(f32), ((0, 0), (0, 5)))
    csh = jnp.pad(p['cons_shift'], (0, 3)).reshape(1, 8)
    csc = jnp.pad(p['cons_scale'], (0, 3), constant_values=1.0).reshape(1, 8)
    vsh = jnp.pad(p['var_shift'], (0, 5)).reshape(1, 24)
    vsc = jnp.pad(p['var_scale'], (0, 5), constant_values=1.0).reshape(1, 24)
    cw1 = jnp.pad(p['cons_W1'], ((0, 3), (0, 0)))
    vw1 = jnp.pad(p['var_W1'], ((0, 5), (0, 0)))

    fs1 = p['v2c_fscale']
    fs2 = p['c2v_fscale']
    wl1 = p['v2c_Wl'] * fs1
    bl1 = (p['v2c_bl'] * fs1).reshape(1, EMB)
    wr1 = p['v2c_Wr'] * fs1
    wl2 = p['c2v_Wl'] * fs2
    bl2 = (p['c2v_bl'] * fs2).reshape(1, EMB)
    wr2 = p['c2v_Wr'] * fs2
    we1 = p['v2c_We'][0]
    we2 = p['c2v_We'][0]
    wfps1 = p['v2c_Wf'] * p['v2c_pscale']
    bfps1 = (p['v2c_bf'] * p['v2c_pscale']).reshape(1, EMB)
    wfps2 = p['c2v_Wf'] * p['c2v_pscale']
    bfps2 = (p['c2v_bf'] * p['c2v_pscale']).reshape(1, EMB)
    es = p['edge_scale'][0]
    esh = p['edge_shift'][0]
    sc1 = (es * fs1).reshape(1, 1)
    sh1 = (esh * es * fs1).reshape(1, 1)
    sc2 = (es * fs2).reshape(1, 1)
    sh2 = (esh * es * fs2).reshape(1, 1)

    erows = E_PAD // 128
    ef1_2d, ef2_2d = pl.pallas_call(
        _edge_body,
        grid=(1,),
        in_specs=[pl.BlockSpec((erows, 128), lambda i: (0, 0)),
                  _smem11(), _smem11(), _smem11(), _smem11()],
        out_specs=[pl.BlockSpec((erows, 128), lambda i: (0, 0))] * 2,
        out_shape=[jax.ShapeDtypeStruct((erows, 128), f32)] * 2,
    )(e2d, sh1, sc1, sh2, sc2)
    ef1 = ef1_2d.reshape(E_PAD)
    ef2 = ef2_2d.reshape(E_PAD)

    def _pack(d, ef):
        parts = [g0, g1, d, lax.bitcast_convert_type(ef, jnp.int32)]
        a = jnp.stack([x.reshape(NSUB, NCHUNK, CH) for x in parts], axis=2)
        return a.reshape(NSUB * NCHUNK, 4, CH)

    pk1 = _pack(d0, ef1)
    pk2 = _pack(d1, ef2)

    cnt0, cnt1 = _sc_count_pass(d0, d1)

    R1 = 2000
    G1 = N_NODE // R1
    row_spec1 = lambda w: pl.BlockSpec((R1, w), lambda i: (i, 0))
    cons, var, tl1, tr1, tr2 = pl.pallas_call(
        _phase1_body,
        grid=(G1,),
        in_specs=[row_spec1(8), row_spec1(24),
                  _full((1, 8)), _full((1, 8)), _full((8, EMB)),
                  _full((1, EMB)), _full((EMB, EMB)), _full((1, EMB)),
                  _full((1, 24)), _full((1, 24)), _full((24, EMB)),
                  _full((1, EMB)), _full((EMB, EMB)), _full((1, EMB)),
                  _full((EMB, EMB)), _full((1, EMB)), _full((EMB, EMB)),
                  _full((EMB, EMB))],
        out_specs=[row_spec1(EMB)] * 5,
        out_shape=[jax.ShapeDtypeStruct((N_NODE, EMB), f32)] * 5,
    )(cf, vf, csh, csc, cw1, p['cons_b1'].reshape(1, EMB), p['cons_W2'],
      p['cons_b2'].reshape(1, EMB), vsh, vsc, vw1,
      p['var_b1'].reshape(1, EMB), p['var_W2'], p['var_b2'].reshape(1, EMB),
      wl1, bl1, wr1, wr2)

    acc1 = _sc_edge_pass(tl1, tr1, pk1, we1)

    R2 = 1000
    G2 = N_NODE // R2
    acc_spec = pl.BlockSpec((1, R2, EMB), lambda i: (i // (G2 // 2),
                                                     i % (G2 // 2), 0))
    cnt_spec = pl.BlockSpec((1, R2, CNTW), lambda i: (i // (G2 // 2),
                                                      i % (G2 // 2), 0))
    row_spec2 = pl.BlockSpec((R2, EMB), lambda i: (i, 0))
    tl2 = pl.pallas_call(
        _phase2_body,
        grid=(G2,),
        in_specs=[acc_spec, cnt_spec, row_spec2,
                  _full((EMB, EMB)), _full((1, EMB)), _full((EMB, EMB)),
                  _full((EMB, EMB)), _full((1, EMB)), _full((EMB, EMB)),
                  _full((1, EMB)), _full((EMB, EMB)), _full((1, EMB))],
        out_specs=row_spec2,
        out_shape=jax.ShapeDtypeStruct((N_NODE, EMB), f32),
    )(acc1, cnt0, cons, wfps1, bfps1, p['v2c_Wo1'][:EMB], p['v2c_Wo1'][EMB:],
      p['v2c_bo1'].reshape(1, EMB), p['v2c_Wo2'],
      p['v2c_bo2'].reshape(1, EMB), wl2, bl2)

    acc2 = _sc_edge_pass(tl2, tr2, pk2, we2)

    y = pl.pallas_call(
        _phase3_body,
        grid=(G2,),
        in_specs=[acc_spec, cnt_spec, row_spec2,
                  _full((EMB, EMB)), _full((1, EMB)), _full((EMB, EMB)),
                  _full((EMB, EMB)), _full((1, EMB)), _full((EMB, EMB)),
                  _full((1, EMB)), _full((EMB, EMB)), _full((1, EMB)),
                  _full((1, EMB))],
        out_specs=pl.BlockSpec((R2, 1), lambda i: (i, 0)),
        out_shape=jax.ShapeDtypeStruct((N_NODE, 1), f32),
    )(acc2, cnt1, var, wfps2, bfps2, p['c2v_Wo1'][:EMB], p['c2v_Wo1'][EMB:],
      p['c2v_bo1'].reshape(1, EMB), p['c2v_Wo2'],
      p['c2v_bo2'].reshape(1, EMB), p['out_W1'],
      p['out_b1'].reshape(1, EMB), p['out_W2'].reshape(1, EMB))

    return y.reshape(1, -1)


# speedup vs baseline: 1.2487x; 1.2487x over previous
import functools

import jax
import jax.numpy as jnp
from jax import lax
from jax.experimental import pallas as pl
from jax.experimental.pallas import tpu as pltpu
from jax.experimental.pallas import tpu_sc as plsc

N_NODE = 50000
E_RAW = 800000
EMB = 64

NCORES = 2
NSUB = 16
CH = 64
NCHUNK = 784
CCH = 64
NCCHUNK = 784
E_TILE = CH * NCHUNK
E_PAD = E_TILE * NSUB
HALF = N_NODE // NCORES
ACC_ROWS = 25088
ROWS_PER_TILE = ACC_ROWS // NSUB
CNTW = 16

_SC_MESH = plsc.VectorSubcoreMesh(core_axis_name="c", subcore_axis_name="s",
                                  num_cores=NCORES, num_subcores=NSUB)
_SC_PARAMS = pltpu.CompilerParams(use_tc_tiling_on_sc=False,
                                  needs_layout_passes=False)


def _localize(idx_ref, lo, n):
    for j in range(n // 16):
        d = idx_ref[pl.ds(j * 16, 16)] - lo
        ok = (d >= 0) & (d < HALF)
        idx_ref[pl.ds(j * 16, 16)] = jnp.where(ok, d, HALF)


def _sc_edge_body(tl_hbm, tr_hbm, pk_hbm, we_hbm,
                  out_hbm, acc, pkA, pkB, locA0, locA1, locB0, locB1,
                  efA0, efA1, efB0, efB1,
                  glA, grA, glB, grB, jointA, jointB, wev,
                  s_pkA, s_pkB, s_glA, s_grA, s_glB, s_grB, s_scA, s_scB):
    c = lax.axis_index("c")
    s = lax.axis_index("s")
    lo = c * HALF

    def _zero_row(r, _):
        for g in range(EMB // 16):
            jointA[r, pl.ds(g * 16, 16)] = jnp.zeros((16,), jnp.float32)
        return 0
    lax.fori_loop(0, CH, _zero_row, 0)

    row0 = s * ROWS_PER_TILE
    nfull = ROWS_PER_TILE // CH
    rem = ROWS_PER_TILE - nfull * CH
    for k in range(nfull):
        pltpu.sync_copy(jointA, acc.at[pl.ds(row0 + k * CH, CH)])
    if rem:
        pltpu.sync_copy(jointA.at[pl.ds(0, rem)],
                        acc.at[pl.ds(row0 + nfull * CH, rem)])

    pltpu.sync_copy(we_hbm, wev)

    plsc.subcore_barrier()

    pkrow = s * NCHUNK

    def _issue_pk(buf, sem, g):
        pltpu.async_copy(pk_hbm.at[pkrow + g], buf, sem)

    def _wait_pk(buf, sem):
        pltpu.make_async_copy(pk_hbm.at[0], buf, sem).wait()

    def _prep(pk, loc, ef):
        for j in range(CH // 16):
            d = pk[2, pl.ds(j * 16, 16)] - lo
            ok = (d >= 0) & (d < HALF)
            loc[pl.ds(j * 16, 16)] = jnp.where(ok, d, HALF)
            ef[pl.ds(j * 16, 16)] = plsc.bitcast(pk[3, pl.ds(j * 16, 16)],
                                                 jnp.float32)

    def _start_gathers(pk, gl, gr, sg1, sg2):
        pltpu.async_copy(tl_hbm.at[pk.at[0]], gl, sg1)
        pltpu.async_copy(tr_hbm.at[pk.at[1]], gr, sg2)

    def _wait_gathers(pk, gl, gr, sg1, sg2):
        pltpu.make_async_copy(tl_hbm.at[pk.at[0]], gl, sg1).wait()
        pltpu.make_async_copy(tr_hbm.at[pk.at[1]], gr, sg2).wait()

    def _compute(gl, gr, ef, joint):
        def _edge16(j, carry):
            efvec = ef[pl.ds(j * 16, 16)]
            ws = tuple(wev[pl.ds(g * 16, 16)] for g in range(EMB // 16))
            for k in range(16):
                e = j * 16 + k
                efs = efvec[k]
                for g, wg in enumerate(ws):
                    v = gl[e, pl.ds(g * 16, 16)] + gr[e, pl.ds(g * 16, 16)]
                    v = jnp.maximum(v + efs * wg, 0.0)
                    joint[e, pl.ds(g * 16, 16)] = v
            return carry
        lax.fori_loop(0, CH // 16, _edge16, 0)

    _issue_pk(pkA, s_pkA, 0)
    _issue_pk(pkB, s_pkB, 1)
    _wait_pk(pkA, s_pkA)
    _prep(pkA, locA0, efA0)
    _start_gathers(pkA, glA, grA, s_glA, s_grA)
    _wait_pk(pkB, s_pkB)
    _prep(pkB, locB0, efB0)
    _start_gathers(pkB, glB, grB, s_glB, s_grB)

    def _half(g, first, pk, s_pk, loc, loc_nxt, ef, ef_nxt, gl, gr, joint,
              sg1, sg2, s_sc):
        _wait_gathers(pk, gl, gr, sg1, sg2)

        @pl.when(g + 2 < NCHUNK)
        def _():
            _issue_pk(pk, s_pk, g + 2)

        @pl.when(jnp.logical_not(first))
        def _():
            pltpu.make_async_copy(joint, acc.at[loc], s_sc).wait()

        _compute(gl, gr, ef, joint)
        pltpu.async_copy(joint, acc.at[loc], s_sc, add=True)

        @pl.when(g + 2 < NCHUNK)
        def _():
            _wait_pk(pk, s_pk)
            _prep(pk, loc_nxt, ef_nxt)
            _start_gathers(pk, gl, gr, sg1, sg2)

    def _quad(k, carry):
        g0 = 4 * k
        _half(g0, k == 0, pkA, s_pkA, locA0, locA1, efA0, efA1,
              glA, grA, jointA, s_glA, s_grA, s_scA)
        _half(g0 + 1, k == 0, pkB, s_pkB, locB0, locB1, efB0, efB1,
              glB, grB, jointB, s_glB, s_grB, s_scB)
        _half(g0 + 2, k < 0, pkA, s_pkA, locA1, locA0, efA1, efA0,
              glA, grA, jointA, s_glA, s_grA, s_scA)
        _half(g0 + 3, k < 0, pkB, s_pkB, locB1, locB0, efB1, efB0,
              glB, grB, jointB, s_glB, s_grB, s_scB)
        return carry

    lax.fori_loop(0, NCHUNK // 4, _quad, 0)

    pltpu.make_async_copy(jointA, acc.at[locA1], s_scA).wait()
    pltpu.make_async_copy(jointB, acc.at[locB1], s_scB).wait()

    plsc.subcore_barrier()
    pltpu.sync_copy(acc.at[pl.ds(row0, ROWS_PER_TILE)],
                    out_hbm.at[c, pl.ds(row0, ROWS_PER_TILE)])


_sc_edge_pass = functools.partial(
    pl.kernel,
    out_type=jax.ShapeDtypeStruct((NCORES, ACC_ROWS, EMB), jnp.float32),
    mesh=_SC_MESH,
    compiler_params=_SC_PARAMS,
    scratch_types=[
        pltpu.VMEM_SHARED((ACC_ROWS, EMB), jnp.float32),
        pltpu.VMEM((4, CH), jnp.int32),
        pltpu.VMEM((4, CH), jnp.int32),
        pltpu.VMEM((CH,), jnp.int32),
        pltpu.VMEM((CH,), jnp.int32),
        pltpu.VMEM((CH,), jnp.int32),
        pltpu.VMEM((CH,), jnp.int32),
        pltpu.VMEM((CH,), jnp.float32),
        pltpu.VMEM((CH,), jnp.float32),
        pltpu.VMEM((CH,), jnp.float32),
        pltpu.VMEM((CH,), jnp.float32),
        pltpu.VMEM((CH, EMB), jnp.float32),
        pltpu.VMEM((CH, EMB), jnp.float32),
        pltpu.VMEM((CH, EMB), jnp.float32),
        pltpu.VMEM((CH, EMB), jnp.float32),
        pltpu.VMEM((CH, EMB), jnp.float32),
        pltpu.VMEM((CH, EMB), jnp.float32),
        pltpu.VMEM((EMB,), jnp.float32),
        pltpu.SemaphoreType.DMA,
        pltpu.SemaphoreType.DMA,
        pltpu.SemaphoreType.DMA,
        pltpu.SemaphoreType.DMA,
        pltpu.SemaphoreType.DMA,
        pltpu.SemaphoreType.DMA,
        pltpu.SemaphoreType.DMA,
        pltpu.SemaphoreType.DMA,
    ],
)(_sc_edge_body)


def _sc_count_body(d0_hbm, d1_hbm, out0_hbm, out1_hbm, cnt0, cnt1,
                   l0A, l1A, l0B, l1B, ones, s0A, s1A, s0B, s1B):
    c = lax.axis_index("c")
    s = lax.axis_index("s")
    lo = c * HALF

    one_vec = jnp.where(lax.iota(jnp.int32, 16) == 0,
                        jnp.float32(1.0), jnp.float32(0.0))
    zero_vec = jnp.zeros((16,), jnp.float32)

    def _init_row(r, _):
        ones[r, pl.ds(0, 16)] = one_vec
        return 0
    lax.fori_loop(0, CCH, _init_row, 0)

    row0 = s * ROWS_PER_TILE

    def _zrow(r, _):
        ones[r, pl.ds(0, 16)] = zero_vec
        return 0
    lax.fori_loop(0, CCH, _zrow, 0)
    nfull = ROWS_PER_TILE // CCH
    rem = ROWS_PER_TILE - nfull * CCH
    for acc in (cnt0, cnt1):
        for k in range(nfull):
            pltpu.sync_copy(ones, acc.at[pl.ds(row0 + k * CCH, CCH)])
        pltpu.sync_copy(ones.at[pl.ds(0, rem)],
                        acc.at[pl.ds(row0 + nfull * CCH, rem)])
    lax.fori_loop(0, CCH, _init_row, 0)

    plsc.subcore_barrier()

    ebase = s * E_TILE

    def _issue(g, l0, l1, sem0, sem1):
        base = ebase + g * CCH
        pltpu.async_copy(d0_hbm.at[pl.ds(base, CCH)], l0, sem0)
        pltpu.async_copy(d1_hbm.at[pl.ds(base, CCH)], l1, sem1)

    def _wait(l0, l1, sem0, sem1):
        pltpu.make_async_copy(d0_hbm.at[pl.ds(0, CCH)], l0, sem0).wait()
        pltpu.make_async_copy(d1_hbm.at[pl.ds(0, CCH)], l1, sem1).wait()

    _issue(0, l0A, l1A, s0A, s1A)
    _issue(1, l0B, l1B, s0B, s1B)

    def _half(g, l0, l1, sem0, sem1):
        _wait(l0, l1, sem0, sem1)
        _localize(l0, lo, CCH)
        _localize(l1, lo, CCH)
        pltpu.sync_copy(ones, cnt0.at[l0], add=True)
        pltpu.sync_copy(ones, cnt1.at[l1], add=True)

        @pl.when(g + 2 < NCCHUNK)
        def _():
            _issue(g + 2, l0, l1, sem0, sem1)

    def _pair(k, carry):
        _half(2 * k, l0A, l1A, s0A, s1A)
        _half(2 * k + 1, l0B, l1B, s0B, s1B)
        return carry

    lax.fori_loop(0, NCCHUNK // 2, _pair, 0)

    plsc.subcore_barrier()
    pltpu.sync_copy(cnt0.at[pl.ds(row0, ROWS_PER_TILE)],
                    out0_hbm.at[c, pl.ds(row0, ROWS_PER_TILE)])
    pltpu.sync_copy(cnt1.at[pl.ds(row0, ROWS_PER_TILE)],
                    out1_hbm.at[c, pl.ds(row0, ROWS_PER_TILE)])


_sc_count_pass = functools.partial(
    pl.kernel,
    out_type=(jax.ShapeDtypeStruct((NCORES, ACC_ROWS, CNTW), jnp.float32),
              jax.ShapeDtypeStruct((NCORES, ACC_ROWS, CNTW), jnp.float32)),
    mesh=_SC_MESH,
    compiler_params=_SC_PARAMS,
    scratch_types=[
        pltpu.VMEM_SHARED((ACC_ROWS, CNTW), jnp.float32),
        pltpu.VMEM_SHARED((ACC_ROWS, CNTW), jnp.float32),
        pltpu.VMEM((CCH,), jnp.int32),
        pltpu.VMEM((CCH,), jnp.int32),
        pltpu.VMEM((CCH,), jnp.int32),
        pltpu.VMEM((CCH,), jnp.int32),
        pltpu.VMEM((CCH, CNTW), jnp.float32),
        pltpu.SemaphoreType.DMA,
        pltpu.SemaphoreType.DMA,
        pltpu.SemaphoreType.DMA,
        pltpu.SemaphoreType.DMA,
    ],
)(_sc_count_body)


def _mm(x, w):
    return jnp.dot(x, w, preferred_element_type=jnp.float32)


def _phase1_body(cf, vf, csh, csc, cw1, cb1, cw2, cb2, vsh, vsc, vw1, vb1,
                 vw2, vb2, wl1, bl1, wr1, wr2,
                 cons_o, var_o, tl1_o, tr1_o, tr2_o):
    ch = (cf[...] + csh[...]) * csc[...]
    ch = jnp.maximum(_mm(ch, cw1[...]) + cb1[...], 0.0)
    cons = jnp.maximum(_mm(ch, cw2[...]) + cb2[...], 0.0)
    vh = (vf[...] + vsh[...]) * vsc[...]
    vh = jnp.maximum(_mm(vh, vw1[...]) + vb1[...], 0.0)
    var = jnp.maximum(_mm(vh, vw2[...]) + vb2[...], 0.0)
    cons_o[...] = cons
    var_o[...] = var
    tl1_o[...] = _mm(cons, wl1[...]) + bl1[...]
    tr1_o[...] = _mm(var, wr1[...])
    tr2_o[...] = _mm(var, wr2[...])


def _phase2_body(acc, cnt, cons, wfps, bfps, wo1a, wo1b, bo1, wo2, bo2,
                 wl2, bl2, tl2_o):
    conv = _mm(acc[0], wfps[...]) + cnt[0][:, 0:1] * bfps[...]
    h = jnp.maximum(_mm(conv, wo1a[...]) + _mm(cons[...], wo1b[...])
                    + bo1[...], 0.0)
    cn = jnp.maximum(_mm(h, wo2[...]) + bo2[...], 0.0)
    tl2_o[...] = _mm(cn, wl2[...]) + bl2[...]


def _phase3_body(acc, cnt, var, wfps, bfps, wo1a, wo1b, bo1, wo2, bo2,
                 ow1, ob1, ow2r, y_o):
    conv = _mm(acc[0], wfps[...]) + cnt[0][:, 0:1] * bfps[...]
    h = jnp.maximum(_mm(conv, wo1a[...]) + _mm(var[...], wo1b[...])
                    + bo1[...], 0.0)
    vn = jnp.maximum(_mm(h, wo2[...]) + bo2[...], 0.0)
    z = jnp.maximum(_mm(vn, ow1[...]) + ob1[...], 0.0)
    y_o[...] = jnp.sum(z * ow2r[...], axis=1, keepdims=True)


def _edge_body(e, sh1, sc1, sh2, sc2, ef1_o, ef2_o):
    ev = e[...]
    ef1_o[...] = ev * sc1[0, 0] + sh1[0, 0]
    ef2_o[...] = ev * sc2[0, 0] + sh2[0, 0]


def _full(shape):
    return pl.BlockSpec(shape, lambda i: tuple(0 for _ in shape))


def _smem11():
    return pl.BlockSpec((1, 1), lambda i: (0, 0), memory_space=pltpu.SMEM)


def kernel(constraint_features, edge_indices, edge_features,
           variable_features, n_cons_per_sample, n_vars_per_sample, params):
    p = params
    f32 = jnp.float32

    idx0 = edge_indices[0].astype(jnp.int32)
    idx1 = edge_indices[1].astype(jnp.int32)
    npad = E_PAD - E_RAW
    zpad = jnp.zeros((npad,), jnp.int32)
    gpad = jnp.full((npad,), N_NODE, jnp.int32)
    g0 = jnp.concatenate([idx0, zpad])
    g1 = jnp.concatenate([idx1, zpad])
    d0 = jnp.concatenate([idx0, gpad])
    d1 = jnp.concatenate([idx1, gpad])
    e_flat = jnp.concatenate([edge_features[:, 0].astype(f32),
                              jnp.zeros((npad,), f32)])
    e2d = e_flat.reshape(E_PAD // 128, 128)

    cf = jnp.pad(constraint_features.astype(f32), ((0, 0), (0, 3)))
    vf = jnp.pad(variable_features.astype(f32), ((0, 0), (0, 5)))
    csh = jnp.pad(p['cons_shift'], (0, 3)).reshape(1, 8)
    csc = jnp.pad(p['cons_scale'], (0, 3), constant_values=1.0).reshape(1, 8)
    vsh = jnp.pad(p['var_shift'], (0, 5)).reshape(1, 24)
    vsc = jnp.pad(p['var_scale'], (0, 5), constant_values=1.0).reshape(1, 24)
    cw1 = jnp.pad(p['cons_W1'], ((0, 3), (0, 0)))
    vw1 = jnp.pad(p['var_W1'], ((0, 5), (0, 0)))

    fs1 = p['v2c_fscale']
    fs2 = p['c2v_fscale']
    wl1 = p['v2c_Wl'] * fs1
    bl1 = (p['v2c_bl'] * fs1).reshape(1, EMB)
    wr1 = p['v2c_Wr'] * fs1
    wl2 = p['c2v_Wl'] * fs2
    bl2 = (p['c2v_bl'] * fs2).reshape(1, EMB)
    wr2 = p['c2v_Wr'] * fs2
    we1 = p['v2c_We'][0]
    we2 = p['c2v_We'][0]
    wfps1 = p['v2c_Wf'] * p['v2c_pscale']
    bfps1 = (p['v2c_bf'] * p['v2c_pscale']).reshape(1, EMB)
    wfps2 = p['c2v_Wf'] * p['c2v_pscale']
    bfps2 = (p['c2v_bf'] * p['c2v_pscale']).reshape(1, EMB)
    es = p['edge_scale'][0]
    esh = p['edge_shift'][0]
    sc1 = (es * fs1).reshape(1, 1)
    sh1 = (esh * es * fs1).reshape(1, 1)
    sc2 = (es * fs2).reshape(1, 1)
    sh2 = (esh * es * fs2).reshape(1, 1)

    erows = E_PAD // 128
    ef1_2d, ef2_2d = pl.pallas_call(
        _edge_body,
        grid=(1,),
        in_specs=[pl.BlockSpec((erows, 128), lambda i: (0, 0)),
                  _smem11(), _smem11(), _smem11(), _smem11()],
        out_specs=[pl.BlockSpec((erows, 128), lambda i: (0, 0))] * 2,
        out_shape=[jax.ShapeDtypeStruct((erows, 128), f32)] * 2,
    )(e2d, sh1, sc1, sh2, sc2)
    ef1 = ef1_2d.reshape(E_PAD)
    ef2 = ef2_2d.reshape(E_PAD)

    def _pack(d, ef):
        parts = [g0, g1, d, lax.bitcast_convert_type(ef, jnp.int32)]
        a = jnp.stack([x.reshape(NSUB, NCHUNK, CH) for x in parts], axis=2)
        return a.reshape(NSUB * NCHUNK, 4, CH)

    pk1 = _pack(d0, ef1)
    pk2 = _pack(d1, ef2)

    cnt0, cnt1 = _sc_count_pass(d0, d1)

    R1 = 2000
    G1 = N_NODE // R1
    row_spec1 = lambda w: pl.BlockSpec((R1, w), lambda i: (i, 0))
    cons, var, tl1, tr1, tr2 = pl.pallas_call(
        _phase1_body,
        grid=(G1,),
        in_specs=[row_spec1(8), row_spec1(24),
                  _full((1, 8)), _full((1, 8)), _full((8, EMB)),
                  _full((1, EMB)), _full((EMB, EMB)), _full((1, EMB)),
                  _full((1, 24)), _full((1, 24)), _full((24, EMB)),
                  _full((1, EMB)), _full((EMB, EMB)), _full((1, EMB)),
                  _full((EMB, EMB)), _full((1, EMB)), _full((EMB, EMB)),
                  _full((EMB, EMB))],
        out_specs=[row_spec1(EMB)] * 5,
        out_shape=[jax.ShapeDtypeStruct((N_NODE, EMB), f32)] * 5,
    )(cf, vf, csh, csc, cw1, p['cons_b1'].reshape(1, EMB), p['cons_W2'],
      p['cons_b2'].reshape(1, EMB), vsh, vsc, vw1,
      p['var_b1'].reshape(1, EMB), p['var_W2'], p['var_b2'].reshape(1, EMB),
      wl1, bl1, wr1, wr2)

    acc1 = _sc_edge_pass(tl1, tr1, pk1, we1)

    R2 = 1000
    G2 = N_NODE // R2
    acc_spec = pl.BlockSpec((1, R2, EMB), lambda i: (i // (G2 // 2),
                                                     i % (G2 // 2), 0))
    cnt_spec = pl.BlockSpec((1, R2, CNTW), lambda i: (i // (G2 // 2),
                                                      i % (G2 // 2), 0))
    row_spec2 = pl.BlockSpec((R2, EMB), lambda i: (i, 0))
    tl2 = pl.pallas_call(
        _phase2_body,
        grid=(G2,),
        in_specs=[acc_spec, cnt_spec, row_spec2,
                  _full((EMB, EMB)), _full((1, EMB)), _full((EMB, EMB)),
                  _full((EMB, EMB)), _full((1, EMB)), _full((EMB, EMB)),
                  _full((1, EMB)), _full((EMB, EMB)), _full((1, EMB))],
        out_specs=row_spec2,
        out_shape=jax.ShapeDtypeStruct((N_NODE, EMB), f32),
    )(acc1, cnt0, cons, wfps1, bfps1, p['v2c_Wo1'][:EMB], p['v2c_Wo1'][EMB:],
      p['v2c_bo1'].reshape(1, EMB), p['v2c_Wo2'],
      p['v2c_bo2'].reshape(1, EMB), wl2, bl2)

    acc2 = _sc_edge_pass(tl2, tr2, pk2, we2)

    y = pl.pallas_call(
        _phase3_body,
        grid=(G2,),
        in_specs=[acc_spec, cnt_spec, row_spec2,
                  _full((EMB, EMB)), _full((1, EMB)), _full((EMB, EMB)),
                  _full((EMB, EMB)), _full((1, EMB)), _full((EMB, EMB)),
                  _full((1, EMB)), _full((EMB, EMB)), _full((1, EMB)),
                  _full((1, EMB))],
        out_specs=pl.BlockSpec((R2, 1), lambda i: (i, 0)),
        out_shape=jax.ShapeDtypeStruct((N_NODE, 1), f32),
    )(acc2, cnt1, var, wfps2, bfps2, p['c2v_Wo1'][:EMB], p['c2v_Wo1'][EMB:],
      p['c2v_bo1'].reshape(1, EMB), p['c2v_Wo2'],
      p['c2v_bo2'].reshape(1, EMB), p['out_W1'],
      p['out_b1'].reshape(1, EMB), p['out_W2'].reshape(1, EMB))

    return y.reshape(1, -1)
